# baseline (speedup 1.0000x reference)
import jax
import jax.numpy as jnp
from jax import lax
from jax.experimental import pallas as pl
from jax.experimental.pallas import tpu as pltpu
from jax.experimental.pallas import tpu_sc as plsc

BATCH = 1024
SEQ = 200
D = 128
NW = 32
ROWS = BATCH * SEQ
R_PER_W = ROWS // NW
CHUNK = SEQ
N_CHUNK = R_PER_W // CHUNK
SPLIT = 128
REM = CHUNK - SPLIT
NBUF = 2


def _emb_body(ids_hbm, tok_hbm, pos_hbm, out_hbm,
              idx_v, rows0, rows1, pos_v,
              gsem0, gsem1, ssem0, ssem1):
    rows = (rows0, rows1)
    gsems = (gsem0, gsem1)
    ssems = (ssem0, ssem1)
    wid = lax.axis_index("s") * 2 + lax.axis_index("c")
    base = wid * R_PER_W
    pltpu.sync_copy(ids_hbm.at[pl.ds(base, R_PER_W)], idx_v)
    pltpu.sync_copy(pos_hbm.at[pl.ds(0, SEQ)], pos_v)

    def start(j, p):
        loc = j * CHUNK

        @pl.when(j >= NBUF)
        def _():
            pltpu.make_async_copy(rows[p], out_hbm.at[pl.ds(0, CHUNK)], ssems[p]).wait()

        pltpu.async_copy(tok_hbm.at[idx_v.at[pl.ds(loc, SPLIT)]],
                         rows[p].at[pl.ds(0, SPLIT)], gsems[p])
        pltpu.async_copy(tok_hbm.at[idx_v.at[pl.ds(loc + SPLIT, REM)]],
                         rows[p].at[pl.ds(SPLIT, REM)], gsems[p])

    def finish(j, p):
        loc = j * CHUNK
        pltpu.make_async_copy(tok_hbm.at[idx_v.at[pl.ds(loc, SPLIT)]],
                              rows[p].at[pl.ds(0, SPLIT)], gsems[p]).wait()
        pltpu.make_async_copy(tok_hbm.at[idx_v.at[pl.ds(loc + SPLIT, REM)]],
                              rows[p].at[pl.ds(SPLIT, REM)], gsems[p]).wait()
        rv = rows[p]

        @plsc.parallel_loop(0, CHUNK, unroll=4)
        def add_body(r):
            for c in range(D // 16):
                sl = pl.ds(c * 16, 16)
                rv[r, sl] = rv[r, sl] + pos_v[r, sl]
        pltpu.async_copy(rv, out_hbm.at[pl.ds(base + loc, CHUNK)], ssems[p])

    start(0, 0)

    def body(i, carry):
        for b in range(NBUF):
            j = i * NBUF + b

            @pl.when(j + 1 < N_CHUNK)
            def _():
                start(j + 1, 1 - b)

            finish(j, b)
        return carry

    lax.fori_loop(0, N_CHUNK // NBUF, body, 0)
    for b in range(NBUF):
        pltpu.make_async_copy(rows[b], out_hbm.at[pl.ds(0, CHUNK)], ssems[b]).wait()


@jax.jit
def _run(ids_flat, tok, pos):
    f = pl.kernel(
        _emb_body,
        mesh=plsc.VectorSubcoreMesh(core_axis_name="c", subcore_axis_name="s"),
        out_type=jax.ShapeDtypeStruct((ROWS, D), jnp.float32),
        scratch_types=[
            pltpu.VMEM((R_PER_W,), jnp.int32),
            pltpu.VMEM((CHUNK, D), jnp.float32),
            pltpu.VMEM((CHUNK, D), jnp.float32),
            pltpu.VMEM((SEQ, D), jnp.float32),
            pltpu.SemaphoreType.DMA,
            pltpu.SemaphoreType.DMA,
            pltpu.SemaphoreType.DMA,
            pltpu.SemaphoreType.DMA,
        ],
    )
    return f(ids_flat, tok, pos)


def kernel(token_ids, token_table, pos_table):
    ids_flat = token_ids.reshape(-1).astype(jnp.int32)
    out = _run(ids_flat, token_table, pos_table)
    return out.reshape(BATCH, SEQ, D)


# speedup vs baseline: 1.1517x; 1.1517x over previous
import jax
import jax.numpy as jnp
from jax import lax
from jax.experimental import pallas as pl
from jax.experimental.pallas import tpu as pltpu
from jax.experimental.pallas import tpu_sc as plsc

BATCH = 1024
SEQ = 200
D = 128
NW = 32
ROWS = BATCH * SEQ
R_PER_W = ROWS // NW
CHUNK = SEQ
N_CHUNK = R_PER_W // CHUNK
SPLIT = 128
REM = CHUNK - SPLIT
NBUF = 3
N_MAIN = (N_CHUNK // NBUF) * NBUF


def _emb_body(ids_hbm, tok_hbm, pos_hbm, out_hbm,
              idx_v, rows0, rows1, rows2, pos_v,
              gsem0, gsem1, gsem2, ssem0, ssem1, ssem2):
    rows = (rows0, rows1, rows2)
    gsems = (gsem0, gsem1, gsem2)
    ssems = (ssem0, ssem1, ssem2)
    wid = lax.axis_index("s") * 2 + lax.axis_index("c")
    base = wid * R_PER_W
    pltpu.sync_copy(ids_hbm.at[pl.ds(base, R_PER_W)], idx_v)
    pltpu.sync_copy(pos_hbm.at[pl.ds(0, SEQ)], pos_v)

    def start(j, p):
        loc = j * CHUNK

        @pl.when(j >= NBUF)
        def _():
            pltpu.make_async_copy(rows[p], out_hbm.at[pl.ds(0, CHUNK)], ssems[p]).wait()

        pltpu.async_copy(tok_hbm.at[idx_v.at[pl.ds(loc, SPLIT)]],
                         rows[p].at[pl.ds(0, SPLIT)], gsems[p])
        pltpu.async_copy(tok_hbm.at[idx_v.at[pl.ds(loc + SPLIT, REM)]],
                         rows[p].at[pl.ds(SPLIT, REM)], gsems[p])

    def finish(j, p):
        loc = j * CHUNK
        pltpu.make_async_copy(tok_hbm.at[idx_v.at[pl.ds(loc, SPLIT)]],
                              rows[p].at[pl.ds(0, SPLIT)], gsems[p]).wait()
        pltpu.make_async_copy(tok_hbm.at[idx_v.at[pl.ds(loc + SPLIT, REM)]],
                              rows[p].at[pl.ds(SPLIT, REM)], gsems[p]).wait()
        rv = rows[p]

        @plsc.parallel_loop(0, CHUNK, unroll=4)
        def add_body(r):
            for c in range(D // 16):
                sl = pl.ds(c * 16, 16)
                rv[r, sl] = rv[r, sl] + pos_v[r, sl]

        pltpu.async_copy(rv, out_hbm.at[pl.ds(base + loc, CHUNK)], ssems[p])

    start(0, 0)

    def body(i, carry):
        for b in range(NBUF):
            j = i * NBUF + b

            @pl.when(j + 1 < N_CHUNK)
            def _():
                start(j + 1, (b + 1) % NBUF)

            finish(j, b)
        return carry

    lax.fori_loop(0, N_MAIN // NBUF, body, 0)
    for j in range(N_MAIN, N_CHUNK):
        if j + 1 < N_CHUNK:
            start(j + 1, (j + 1) % NBUF)
        finish(j, j % NBUF)
    for b in range(NBUF):
        pltpu.make_async_copy(rows[b], out_hbm.at[pl.ds(0, CHUNK)], ssems[b]).wait()


@jax.jit
def _run(ids_flat, tok, pos):
    f = pl.kernel(
        _emb_body,
        mesh=plsc.VectorSubcoreMesh(core_axis_name="c", subcore_axis_name="s"),
        out_type=jax.ShapeDtypeStruct((ROWS, D), jnp.float32),
        scratch_types=[
            pltpu.VMEM((R_PER_W,), jnp.int32),
            pltpu.VMEM((CHUNK, D), jnp.float32),
            pltpu.VMEM((CHUNK, D), jnp.float32),
            pltpu.VMEM((CHUNK, D), jnp.float32),
            pltpu.VMEM((SEQ, D), jnp.float32),
            pltpu.SemaphoreType.DMA,
            pltpu.SemaphoreType.DMA,
            pltpu.SemaphoreType.DMA,
            pltpu.SemaphoreType.DMA,
            pltpu.SemaphoreType.DMA,
            pltpu.SemaphoreType.DMA,
        ],
    )
    return f(ids_flat, tok, pos)


def kernel(token_ids, token_table, pos_table):
    ids_flat = token_ids.reshape(-1).astype(jnp.int32)
    out = _run(ids_flat, token_table, pos_table)
    return out.reshape(BATCH, SEQ, D)
